# baseline (speedup 1.0000x reference)
import functools

import jax
import jax.numpy as jnp
from jax import lax
from jax.experimental import pallas as pl
from jax.experimental.pallas import tpu as pltpu
from jax.experimental.pallas import tpu_sc as plsc

_NC = 2
_NS = 16


def kernel(shape_ids, color_ids, edge_index, shape_embed, color_embed,
           W1, b1, W2, b2, Wp, bp):
  f32 = jnp.float32
  N = shape_ids.shape[0]
  E = edge_index.shape[1]
  H = W1.shape[1]
  OUTD = Wp.shape[1]
  EW = shape_embed.shape[1]
  SLABS = 8
  SW = H // SLABS
  BN = 512
  PAD_N = -(-(N + 16) // BN) * BN
  assert PAD_N % (_NS * 8) == 0
  STRIPE = PAD_N // _NS
  DUMMY = PAD_N - 8
  KB = 16
  CH = _NS * 128 * KB
  E_pad = -(-E // CH) * CH
  EROWS = E_pad // 128
  ERT = EROWS // _NS
  NOUT = ERT // KB
  KB_D = 4
  ERT_D = EROWS // (_NS * _NC)
  assert ERT_D % KB_D == 0
  NOUT_D = ERT_D // KB_D

  src = edge_index[0].astype(jnp.int32)
  dst = edge_index[1].astype(jnp.int32)
  src_p = jnp.pad(src, (0, E_pad - E))
  dst_p = jnp.pad(dst, (0, E_pad - E), constant_values=DUMMY)
  srcs = (src_p[None, :] * SLABS
          + jnp.arange(SLABS, dtype=jnp.int32)[:, None]
          ).reshape(SLABS * EROWS, 128)
  srcs = jnp.concatenate(
      [srcs, jnp.zeros((KB, 128), jnp.int32)])
  dst2 = jnp.concatenate(
      [dst_p.reshape(EROWS, 128),
       jnp.full((KB, 128), DUMMY, jnp.int32)])
  zeros16 = jnp.zeros((STRIPE, 16), f32)
  ones16 = jnp.ones((128, 16), f32)

  mesh = plsc.VectorSubcoreMesh(core_axis_name="c", subcore_axis_name="s",
                                num_cores=_NC, num_subcores=_NS)
  sc_params = pltpu.CompilerParams(use_tc_tiling_on_sc=False)

  @functools.partial(
      pl.kernel,
      out_type=jax.ShapeDtypeStruct((_NC * PAD_N, 16), f32),
      mesh=mesh,
      scratch_types=[
          pltpu.VMEM((KB_D, 128), jnp.int32),
          pltpu.VMEM((128, 16), f32),
          pltpu.VMEM_SHARED((PAD_N, 16), f32),
      ],
      compiler_params=sc_params)
  def deg_kernel(dst_hbm, ones_h, zeros_h, deg_out, didx, ones_v, acc):
    c = lax.axis_index("c")
    t = lax.axis_index("s")
    w = c * _NS + t
    pltpu.sync_copy(ones_h, ones_v)
    pltpu.sync_copy(zeros_h, acc.at[pl.ds(t * STRIPE, STRIPE)])
    plsc.subcore_barrier()

    def body(i, carry):
      r0 = w * ERT_D + i * KB_D
      pltpu.sync_copy(dst_hbm.at[pl.ds(r0, KB_D)], didx)
      for j in range(KB_D):
        pltpu.sync_copy(ones_v, acc.at[didx.at[j]], add=True)
      return carry

    lax.fori_loop(0, NOUT_D, body, 0)
    plsc.subcore_barrier()
    pltpu.sync_copy(acc.at[pl.ds(t * STRIPE, STRIPE)],
                    deg_out.at[pl.ds(c * PAD_N + t * STRIPE, STRIPE)])

  @functools.partial(
      pl.kernel,
      out_type=jax.ShapeDtypeStruct((PAD_N, H), f32),
      mesh=mesh,
      scratch_types=[
          pltpu.VMEM((2 * KB, 128), jnp.int32),
          pltpu.VMEM((2 * KB, 128), jnp.int32),
          pltpu.VMEM((KB * 128, SW), f32),
          pltpu.VMEM_SHARED((PAD_N, SW), f32),
          pltpu.SemaphoreType.DMA,
          pltpu.SemaphoreType.DMA,
          pltpu.SemaphoreType.DMA,
      ],
      compiler_params=sc_params)
  def prop_kernel(y_hbm, srcs_hbm, dst_hbm, zeros_h, z_out,
                  sidx, didx, rows, acc, sem, sem2, sem3):
    c = lax.axis_index("c")
    t = lax.axis_index("s")
    for sl in range(SLABS // _NC):
      slab = c * (SLABS // _NC) + sl
      pltpu.sync_copy(zeros_h, acc.at[pl.ds(t * STRIPE, STRIPE)])
      plsc.subcore_barrier()

      def body(i, carry):
        r0 = t * ERT + i * KB
        pltpu.sync_copy(srcs_hbm.at[pl.ds(slab * EROWS + r0, KB)],
                        sidx.at[pl.ds(0, KB)])
        pltpu.sync_copy(dst_hbm.at[pl.ds(r0, KB)],
                        didx.at[pl.ds(0, KB)])
        gathers = [
            pltpu.async_copy(y_hbm.at[sidx.at[j]],
                             rows.at[pl.ds(j * 128, 128)], sem)
            for j in range(KB)
        ]
        scatters = []
        for j in range(KB):
          gathers[j].wait()
          scatters.append(
              pltpu.async_copy(rows.at[pl.ds(j * 128, 128)],
                               acc.at[didx.at[j]], sem2, add=True))
        for j in range(KB):
          scatters[j].wait()
        return carry

      lax.fori_loop(0, NOUT, body, 0)
      plsc.subcore_barrier()
      pltpu.sync_copy(acc.at[pl.ds(t * STRIPE, STRIPE)],
                      z_out.at[pl.ds(t * STRIPE, STRIPE),
                               pl.ds(slab * SW, SW)])
      plsc.subcore_barrier()

  NB = PAD_N // BN

  def tca_body(sid_ref, cid_ref, se_ref, ce_ref, w1_ref, dga_ref, dgb_ref,
               y_ref, dinv_ref):
    sid = sid_ref[0, 0, :].reshape(BN, 1)
    cid = cid_ref[0, 0, :].reshape(BN, 1)
    oh_s = (sid == lax.broadcasted_iota(jnp.int32, (BN, 8), 1)).astype(f32)
    oh_c = (cid == lax.broadcasted_iota(jnp.int32, (BN, 8), 1)).astype(f32)
    a_s = jnp.dot(se_ref[...], w1_ref[...][:EW, :], preferred_element_type=f32)
    a_c = jnp.dot(ce_ref[...], w1_ref[...][EW:, :], preferred_element_type=f32)
    xw = (jnp.dot(oh_s, a_s, preferred_element_type=f32)
          + jnp.dot(oh_c, a_c, preferred_element_type=f32))
    deg = dga_ref[...][:, :1] + dgb_ref[...][:, :1] + 1.0
    dinv = lax.rsqrt(deg)
    y_ref[...] = xw * dinv
    dinv_ref[...] = jnp.broadcast_to(dinv, (BN, 8))

  tca = pl.pallas_call(
      tca_body,
      grid=(NB,),
      in_specs=[
          pl.BlockSpec((1, 1, BN), lambda i: (i, 0, 0)),
          pl.BlockSpec((1, 1, BN), lambda i: (i, 0, 0)),
          pl.BlockSpec((8, EW), lambda i: (0, 0)),
          pl.BlockSpec((8, EW), lambda i: (0, 0)),
          pl.BlockSpec((2 * EW, H), lambda i: (0, 0)),
          pl.BlockSpec((BN, 16), lambda i: (i, 0)),
          pl.BlockSpec((BN, 16), lambda i: (i, 0)),
      ],
      out_specs=[
          pl.BlockSpec((BN, H), lambda i: (i, 0)),
          pl.BlockSpec((BN, 8), lambda i: (i, 0)),
      ],
      out_shape=[
          jax.ShapeDtypeStruct((PAD_N, H), f32),
          jax.ShapeDtypeStruct((PAD_N, 8), f32),
      ],
  )

  def tcb_body(z_ref, y_ref, dinv_ref, b_ref, w_ref, o_ref):
    dinv = dinv_ref[...][:, :1]
    h = jnp.maximum((z_ref[...] + y_ref[...]) * dinv + b_ref[...], 0.0)
    o_ref[...] = jnp.dot(h, w_ref[...], preferred_element_type=f32) * dinv

  tcb = pl.pallas_call(
      tcb_body,
      grid=(NB,),
      in_specs=[
          pl.BlockSpec((BN, H), lambda i: (i, 0)),
          pl.BlockSpec((BN, H), lambda i: (i, 0)),
          pl.BlockSpec((BN, 8), lambda i: (i, 0)),
          pl.BlockSpec((1, H), lambda i: (0, 0)),
          pl.BlockSpec((H, H), lambda i: (0, 0)),
      ],
      out_specs=pl.BlockSpec((BN, H), lambda i: (i, 0)),
      out_shape=jax.ShapeDtypeStruct((PAD_N, H), f32),
  )

  def tcc_body(z_ref, y_ref, dinv_ref, b_ref, wp_ref, bp_ref, o_ref):
    dinv = dinv_ref[...][:, :1]
    h = jnp.maximum((z_ref[...] + y_ref[...]) * dinv + b_ref[...], 0.0)
    o_ref[...] = (jnp.dot(h, wp_ref[...], preferred_element_type=f32)
                  + bp_ref[...])

  tcc = pl.pallas_call(
      tcc_body,
      grid=(NB,),
      in_specs=[
          pl.BlockSpec((BN, H), lambda i: (i, 0)),
          pl.BlockSpec((BN, H), lambda i: (i, 0)),
          pl.BlockSpec((BN, 8), lambda i: (i, 0)),
          pl.BlockSpec((1, H), lambda i: (0, 0)),
          pl.BlockSpec((H, OUTD), lambda i: (0, 0)),
          pl.BlockSpec((1, OUTD), lambda i: (0, 0)),
      ],
      out_specs=pl.BlockSpec((BN, OUTD), lambda i: (i, 0)),
      out_shape=jax.ShapeDtypeStruct((PAD_N, OUTD), f32),
  )

  sid3 = jnp.pad(shape_ids.astype(jnp.int32), (0, PAD_N - N)).reshape(NB, 1, BN)
  cid3 = jnp.pad(color_ids.astype(jnp.int32), (0, PAD_N - N)).reshape(NB, 1, BN)
  se_p = jnp.pad(shape_embed, ((0, 8 - shape_embed.shape[0]), (0, 0)))
  ce_p = jnp.pad(color_embed, ((0, 8 - color_embed.shape[0]), (0, 0)))

  deg_out = deg_kernel(dst2, ones16, zeros16)
  dg0 = deg_out[:PAD_N]
  dg1 = deg_out[PAD_N:]
  y1, dinv = tca(sid3, cid3, se_p, ce_p, W1, dg0, dg1)

  z1 = prop_kernel(y1.reshape(SLABS * PAD_N, SW), srcs, dst2, zeros16)
  y2 = tcb(z1, y1, dinv, b1.reshape(1, H), W2)
  z2 = prop_kernel(y2.reshape(SLABS * PAD_N, SW), srcs, dst2, zeros16)
  out = tcc(z2, y2, dinv, b2.reshape(1, H), Wp, bp.reshape(1, OUTD))
  return out[:N]


# speedup vs baseline: 1.8256x; 1.8256x over previous
import functools

import jax
import jax.numpy as jnp
from jax import lax
from jax.experimental import pallas as pl
from jax.experimental.pallas import tpu as pltpu
from jax.experimental.pallas import tpu_sc as plsc

_NC = 2
_NS = 16


def kernel(shape_ids, color_ids, edge_index, shape_embed, color_embed,
           W1, b1, W2, b2, Wp, bp):
  f32 = jnp.float32
  N = shape_ids.shape[0]
  E = edge_index.shape[1]
  H = W1.shape[1]
  OUTD = Wp.shape[1]
  EW = shape_embed.shape[1]
  SLABS = 8
  SW = H // SLABS
  BN = 512
  PAD_N = -(-(N + 16) // BN) * BN
  assert PAD_N % (_NS * 8) == 0
  STRIPE = PAD_N // _NS
  DUMMY = PAD_N - 8
  KB = 8
  CH = _NS * 128 * KB
  E_pad = -(-E // CH) * CH
  EROWS = E_pad // 128
  ERT = EROWS // _NS
  NOUT = ERT // KB
  KB_D = 4
  ERT_D = EROWS // (_NS * _NC)
  assert ERT_D % KB_D == 0
  NOUT_D = ERT_D // KB_D

  src = edge_index[0].astype(jnp.int32)
  dst = edge_index[1].astype(jnp.int32)
  src_p = jnp.pad(src, (0, E_pad - E))
  dst_p = jnp.pad(dst, (0, E_pad - E), constant_values=DUMMY)
  srcs = (src_p[None, :] * SLABS
          + jnp.arange(SLABS, dtype=jnp.int32)[:, None]
          ).reshape(SLABS * EROWS, 128)
  srcs = jnp.concatenate(
      [srcs, jnp.zeros((KB, 128), jnp.int32)])
  dst2 = jnp.concatenate(
      [dst_p.reshape(EROWS, 128),
       jnp.full((KB, 128), DUMMY, jnp.int32)])
  zeros16 = jnp.zeros((STRIPE, 16), f32)
  ones16 = jnp.ones((128, 16), f32)

  mesh = plsc.VectorSubcoreMesh(core_axis_name="c", subcore_axis_name="s",
                                num_cores=_NC, num_subcores=_NS)
  sc_params = pltpu.CompilerParams(use_tc_tiling_on_sc=False)

  @functools.partial(
      pl.kernel,
      out_type=jax.ShapeDtypeStruct((PAD_N, H), f32),
      mesh=mesh,
      scratch_types=[
          pltpu.VMEM((KB_D, 128), jnp.int32),
          pltpu.VMEM((128, 16), f32),
          pltpu.VMEM_SHARED((PAD_N, 16), f32),
      ],
      compiler_params=sc_params)
  def deg_kernel(dst_hbm, ones_h, zeros_h, deg_out, didx, ones_v, acc):
    c = lax.axis_index("c")
    t = lax.axis_index("s")
    w = c * _NS + t
    pltpu.sync_copy(ones_h, ones_v)
    pltpu.sync_copy(zeros_h, acc.at[pl.ds(t * STRIPE, STRIPE)])
    plsc.subcore_barrier()

    def body(i, carry):
      r0 = w * ERT_D + i * KB_D
      pltpu.sync_copy(dst_hbm.at[pl.ds(r0, KB_D)], didx)
      for j in range(KB_D):
        pltpu.sync_copy(ones_v, acc.at[didx.at[j]], add=True)
      return carry

    lax.fori_loop(0, NOUT_D, body, 0)
    plsc.subcore_barrier()
    pltpu.sync_copy(acc.at[pl.ds(t * STRIPE, STRIPE)],
                    deg_out.at[pl.ds(t * STRIPE, STRIPE),
                               pl.ds(c * 16, 16)])

  @functools.partial(
      pl.kernel,
      out_type=jax.ShapeDtypeStruct((PAD_N, H), f32),
      mesh=mesh,
      scratch_types=[
          pltpu.VMEM((2 * KB, 128), jnp.int32),
          pltpu.VMEM((2 * KB, 128), jnp.int32),
          pltpu.VMEM((2 * KB * 128, SW), f32),
          pltpu.VMEM_SHARED((PAD_N, SW), f32),
          pltpu.SemaphoreType.DMA,
          pltpu.SemaphoreType.DMA,
          pltpu.SemaphoreType.DMA,
      ],
      compiler_params=sc_params)
  def prop_kernel(y_hbm, srcs_hbm, dst_hbm, zeros_h, z_out,
                  sidx, didx, rows, acc, sem, sem2, sem3):
    c = lax.axis_index("c")
    t = lax.axis_index("s")
    for sl in range(SLABS // _NC):
      slab = c * (SLABS // _NC) + sl
      pltpu.sync_copy(zeros_h, acc.at[pl.ds(t * STRIPE, STRIPE)])
      plsc.subcore_barrier()

      pltpu.sync_copy(srcs_hbm.at[pl.ds(slab * EROWS + t * ERT, KB)],
                      sidx.at[pl.ds(0, KB)])
      pltpu.sync_copy(dst_hbm.at[pl.ds(t * ERT, KB)],
                      didx.at[pl.ds(0, KB)])

      def body(i, carry):
        par = jnp.bitwise_and(i, 1)
        base = par * KB
        nbase = (1 - par) * KB
        rbase = par * KB * 128
        r1 = t * ERT + (i + 1) * KB
        gathers = [
            pltpu.async_copy(y_hbm.at[sidx.at[base + j]],
                             rows.at[pl.ds(rbase + j * 128, 128)], sem)
            for j in range(KB)
        ]
        for j in range(KB):
          pltpu.make_async_copy(rows.at[pl.ds(j * 128, 128)],
                                acc.at[didx.at[j]], sem2).wait()
        ld_s = pltpu.async_copy(srcs_hbm.at[pl.ds(slab * EROWS + r1, KB)],
                                sidx.at[pl.ds(nbase, KB)], sem3)
        ld_d = pltpu.async_copy(dst_hbm.at[pl.ds(r1, KB)],
                                didx.at[pl.ds(nbase, KB)], sem3)
        for j in range(KB):
          gathers[j].wait()
          pltpu.async_copy(rows.at[pl.ds(rbase + j * 128, 128)],
                           acc.at[didx.at[base + j]], sem2, add=True)
        ld_s.wait()
        ld_d.wait()
        return carry

      pltpu.sync_copy(zeros_h.at[pl.ds(0, 128)],
                      rows.at[pl.ds(KB * 128, 128)])
      for j in range(KB):
        pltpu.async_copy(rows.at[pl.ds(KB * 128, 128)],
                         acc.at[didx.at[j]], sem2, add=True)
      lax.fori_loop(0, NOUT, body, 0)
      for j in range(KB):
        pltpu.make_async_copy(rows.at[pl.ds(j * 128, 128)],
                              acc.at[didx.at[j]], sem2).wait()
      plsc.subcore_barrier()
      pltpu.sync_copy(acc.at[pl.ds(t * STRIPE, STRIPE)],
                      z_out.at[pl.ds(t * STRIPE, STRIPE),
                               pl.ds(slab * SW, SW)])
      plsc.subcore_barrier()

  NB = PAD_N // BN

  def tca_body(sid_ref, cid_ref, se_ref, ce_ref, w1_ref, dg_ref,
               y_ref, dinv_ref):
    sid = sid_ref[0, 0, :].reshape(BN, 1)
    cid = cid_ref[0, 0, :].reshape(BN, 1)
    oh_s = (sid == lax.broadcasted_iota(jnp.int32, (BN, 8), 1)).astype(f32)
    oh_c = (cid == lax.broadcasted_iota(jnp.int32, (BN, 8), 1)).astype(f32)
    a_s = jnp.dot(se_ref[...], w1_ref[...][:EW, :], preferred_element_type=f32)
    a_c = jnp.dot(ce_ref[...], w1_ref[...][EW:, :], preferred_element_type=f32)
    xw = (jnp.dot(oh_s, a_s, preferred_element_type=f32)
          + jnp.dot(oh_c, a_c, preferred_element_type=f32))
    d = dg_ref[...]
    deg = d[:, 0:1] + d[:, 16:17] + 1.0
    dinv = lax.rsqrt(deg)
    y_ref[...] = xw * dinv
    dinv_ref[...] = jnp.broadcast_to(dinv, (BN, 8))

  tca = pl.pallas_call(
      tca_body,
      grid=(NB,),
      in_specs=[
          pl.BlockSpec((1, 1, BN), lambda i: (i, 0, 0)),
          pl.BlockSpec((1, 1, BN), lambda i: (i, 0, 0)),
          pl.BlockSpec((8, EW), lambda i: (0, 0)),
          pl.BlockSpec((8, EW), lambda i: (0, 0)),
          pl.BlockSpec((2 * EW, H), lambda i: (0, 0)),
          pl.BlockSpec((BN, H), lambda i: (i, 0)),
      ],
      out_specs=[
          pl.BlockSpec((BN, H), lambda i: (i, 0)),
          pl.BlockSpec((BN, 8), lambda i: (i, 0)),
      ],
      out_shape=[
          jax.ShapeDtypeStruct((PAD_N, H), f32),
          jax.ShapeDtypeStruct((PAD_N, 8), f32),
      ],
  )

  def tcb_body(z_ref, y_ref, dinv_ref, b_ref, w_ref, o_ref):
    dinv = dinv_ref[...][:, :1]
    h = jnp.maximum((z_ref[...] + y_ref[...]) * dinv + b_ref[...], 0.0)
    o_ref[...] = jnp.dot(h, w_ref[...], preferred_element_type=f32) * dinv

  tcb = pl.pallas_call(
      tcb_body,
      grid=(NB,),
      in_specs=[
          pl.BlockSpec((BN, H), lambda i: (i, 0)),
          pl.BlockSpec((BN, H), lambda i: (i, 0)),
          pl.BlockSpec((BN, 8), lambda i: (i, 0)),
          pl.BlockSpec((1, H), lambda i: (0, 0)),
          pl.BlockSpec((H, H), lambda i: (0, 0)),
      ],
      out_specs=pl.BlockSpec((BN, H), lambda i: (i, 0)),
      out_shape=jax.ShapeDtypeStruct((PAD_N, H), f32),
  )

  def tcc_body(z_ref, y_ref, dinv_ref, b_ref, wp_ref, bp_ref, o_ref):
    dinv = dinv_ref[...][:, :1]
    h = jnp.maximum((z_ref[...] + y_ref[...]) * dinv + b_ref[...], 0.0)
    o_ref[...] = (jnp.dot(h, wp_ref[...], preferred_element_type=f32)
                  + bp_ref[...])

  tcc = pl.pallas_call(
      tcc_body,
      grid=(NB,),
      in_specs=[
          pl.BlockSpec((BN, H), lambda i: (i, 0)),
          pl.BlockSpec((BN, H), lambda i: (i, 0)),
          pl.BlockSpec((BN, 8), lambda i: (i, 0)),
          pl.BlockSpec((1, H), lambda i: (0, 0)),
          pl.BlockSpec((H, OUTD), lambda i: (0, 0)),
          pl.BlockSpec((1, OUTD), lambda i: (0, 0)),
      ],
      out_specs=pl.BlockSpec((BN, OUTD), lambda i: (i, 0)),
      out_shape=jax.ShapeDtypeStruct((PAD_N, OUTD), f32),
  )

  sid3 = jnp.pad(shape_ids.astype(jnp.int32), (0, PAD_N - N)).reshape(NB, 1, BN)
  cid3 = jnp.pad(color_ids.astype(jnp.int32), (0, PAD_N - N)).reshape(NB, 1, BN)
  se_p = jnp.pad(shape_embed, ((0, 8 - shape_embed.shape[0]), (0, 0)))
  ce_p = jnp.pad(color_embed, ((0, 8 - color_embed.shape[0]), (0, 0)))

  deg_out = deg_kernel(dst2, ones16, zeros16)
  y1, dinv = tca(sid3, cid3, se_p, ce_p, W1, deg_out)

  z1 = prop_kernel(y1.reshape(SLABS * PAD_N, SW), srcs, dst2, zeros16)
  y2 = tcb(z1, y1, dinv, b1.reshape(1, H), W2)
  z2 = prop_kernel(y2.reshape(SLABS * PAD_N, SW), srcs, dst2, zeros16)
  out = tcc(z2, y2, dinv, b2.reshape(1, H), Wp, bp.reshape(1, OUTD))
  return out[:N]
